# Initial kernel scaffold; baseline (speedup 1.0000x reference)
import functools

import jax
import jax.numpy as jnp
from jax import lax
from jax.experimental import pallas as pl
from jax.experimental.pallas import tpu as pltpu
from jax.experimental.pallas import tpu_sc as plsc

NC, NS = 2, 16
NW = NC * NS
CHUNK = 512


@functools.lru_cache(maxsize=None)
def _build(n_idx, vocab, d_model):
    assert n_idx % (NW * CHUNK) == 0
    b_per_w = n_idx // NW
    n_chunks = b_per_w // CHUNK

    mesh = plsc.VectorSubcoreMesh(core_axis_name="c", subcore_axis_name="s")

    @functools.partial(
        pl.kernel,
        out_type=jax.ShapeDtypeStruct((n_idx, d_model), jnp.float32),
        mesh=mesh,
        scratch_types=[
            pltpu.VMEM((CHUNK,), jnp.int32),
            pltpu.VMEM((CHUNK, d_model), jnp.float32),
            pltpu.SemaphoreType.DMA,
        ],
    )
    def embed(table_hbm, idx_hbm, out_hbm, idx_v, rows_v, sem):
        wid = lax.axis_index("s") * NC + lax.axis_index("c")
        base = wid * b_per_w

        def step(g, carry):
            off = base + g * CHUNK
            pltpu.sync_copy(idx_hbm.at[pl.ds(off, CHUNK)], idx_v)
            pltpu.async_copy(table_hbm.at[idx_v], rows_v, sem).wait()
            pltpu.sync_copy(rows_v, out_hbm.at[pl.ds(off, CHUNK)])
            return carry

        lax.fori_loop(0, n_chunks, step, 0)

    return embed


def kernel(x, table):
    b, h = x.shape
    vocab, d_model = table.shape
    flat = x.reshape(b * h).astype(jnp.int32)
    out = _build(b * h, vocab, d_model)(table, flat)
    return out.reshape(b, h, d_model)


# speedup vs baseline: 1.7967x; 1.7967x over previous
import functools

import jax
import jax.numpy as jnp
from jax import lax
from jax.experimental import pallas as pl
from jax.experimental.pallas import tpu as pltpu
from jax.experimental.pallas import tpu_sc as plsc

NC, NS = 2, 16
NW = NC * NS
CHUNK = 512


@functools.lru_cache(maxsize=None)
def _build(n_idx, vocab, d_model):
    assert n_idx % (NW * CHUNK) == 0
    b_per_w = n_idx // NW
    n_chunks = b_per_w // CHUNK

    mesh = plsc.VectorSubcoreMesh(core_axis_name="c", subcore_axis_name="s")

    @functools.partial(
        pl.kernel,
        out_type=jax.ShapeDtypeStruct((n_idx, d_model), jnp.float32),
        mesh=mesh,
        scratch_types=[
            pltpu.VMEM((CHUNK,), jnp.int32),
            pltpu.VMEM((CHUNK, d_model), jnp.float32),
            pltpu.SemaphoreType.DMA,
        ],
        compiler_params=pltpu.CompilerParams(use_tc_tiling_on_sc=False),
    )
    def embed(table_hbm, idx_hbm, out_hbm, idx_v, rows_v, sem):
        wid = lax.axis_index("s") * NC + lax.axis_index("c")
        base = wid * b_per_w

        def step(g, carry):
            off = base + g * CHUNK
            pltpu.sync_copy(idx_hbm.at[pl.ds(off, CHUNK)], idx_v)
            pltpu.async_copy(table_hbm.at[idx_v], rows_v, sem).wait()
            pltpu.sync_copy(rows_v, out_hbm.at[pl.ds(off, CHUNK)])
            return carry

        lax.fori_loop(0, n_chunks, step, 0)

    return embed


def kernel(x, table):
    b, h = x.shape
    vocab, d_model = table.shape
    flat = x.reshape(b * h).astype(jnp.int32)
    out = _build(b * h, vocab, d_model)(table, flat)
    return out.reshape(b, h, d_model)


# speedup vs baseline: 1.8735x; 1.0427x over previous
import functools

import jax
import jax.numpy as jnp
from jax import lax
from jax.experimental import pallas as pl
from jax.experimental.pallas import tpu as pltpu
from jax.experimental.pallas import tpu_sc as plsc

NC, NS = 2, 16
NW = NC * NS
CHUNK = 800


@functools.lru_cache(maxsize=None)
def _build(n_idx, vocab, d_model):
    assert n_idx % (NW * CHUNK) == 0
    b_per_w = n_idx // NW
    n_chunks = b_per_w // CHUNK
    assert n_chunks % 2 == 0 and n_chunks >= 4

    mesh = plsc.VectorSubcoreMesh(core_axis_name="c", subcore_axis_name="s")

    @functools.partial(
        pl.kernel,
        out_type=jax.ShapeDtypeStruct((n_idx, d_model), jnp.float32),
        mesh=mesh,
        scratch_types=[
            pltpu.VMEM((2, CHUNK), jnp.int32),
            pltpu.VMEM((2, CHUNK, d_model), jnp.float32),
        ] + [pltpu.SemaphoreType.DMA] * 6,
        compiler_params=pltpu.CompilerParams(use_tc_tiling_on_sc=False),
    )
    def embed(table_hbm, idx_hbm, out_hbm, idx_v, rows_v, si0, si1, sg0, sg1, so0, so1):
        sem_i = (si0, si1)
        sem_g = (sg0, sg1)
        sem_o = (so0, so1)
        wid = lax.axis_index("s") * NC + lax.axis_index("c")
        base = wid * b_per_w

        def idx_cp(g, s):
            return pltpu.make_async_copy(
                idx_hbm.at[pl.ds(base + g * CHUNK, CHUNK)], idx_v.at[s], sem_i[s])

        def gat_cp(s):
            return pltpu.make_async_copy(
                table_hbm.at[idx_v.at[s]], rows_v.at[s], sem_g[s])

        def out_cp(g, s):
            return pltpu.make_async_copy(
                rows_v.at[s], out_hbm.at[pl.ds(base + g * CHUNK, CHUNK)], sem_o[s])

        idx_cp(0, 0).start()
        idx_cp(1, 1).start()

        def body(k, carry):
            for b in (0, 1):
                g = 2 * k + b
                s = b

                @pl.when(g >= 2)
                def _():
                    out_cp(g - 2, s).wait()

                idx_cp(g, s).wait()
                gat_cp(s).start()

                @pl.when(g >= 1)
                def _():
                    gat_cp(1 - s).wait()
                    out_cp(g - 1, 1 - s).start()

                @pl.when((g >= 1) & (g + 1 < n_chunks))
                def _():
                    idx_cp(g + 1, 1 - s).start()
            return carry

        lax.fori_loop(0, n_chunks // 2, body, 0)

        s_last = (n_chunks - 1) % 2
        gat_cp(s_last).wait()
        out_cp(n_chunks - 1, s_last).start()
        out_cp(n_chunks - 2, 1 - s_last).wait()
        out_cp(n_chunks - 1, s_last).wait()

    return embed


def kernel(x, table):
    b, h = x.shape
    vocab, d_model = table.shape
    flat = x.reshape(b * h).astype(jnp.int32)
    out = _build(b * h, vocab, d_model)(table, flat)
    return out.reshape(b, h, d_model)
